# baseline (speedup 1.0000x reference)
import jax
import jax.numpy as jnp
from jax.experimental import pallas as pl
from jax.experimental.pallas import tpu as pltpu


def _body(sidx, coefs, x_ref, lpe_ref, gpe_ref, o_ref):
    bt = pl.program_id(1)
    n_bt = pl.num_programs(1)
    c = coefs[bt]
    lg = coefs[n_bt]
    o_ref[0] = x_ref[0] + lpe_ref[...] * lg + gpe_ref[0, 0] * c


def kernel(x, aspect_ratio, local_pe, global_pe, gate):
    bsz, n_tiles, n_tokens, embed_dim = x.shape
    bt_total = bsz * n_tiles

    g = jnp.tanh(gate)[0]
    t = jnp.arange(n_tiles, dtype=jnp.int32)
    h = aspect_ratio[:, 0:1]
    w = aspect_ratio[:, 1:2]
    w_safe = jnp.maximum(w, 1)
    row = (t[None, :] // w_safe).astype(jnp.int32)
    col = (t[None, :] % w_safe).astype(jnp.int32)
    mask = t[None, :] < (h * w)
    row = jnp.where(mask, row, 0).reshape(bt_total)
    col = jnp.where(mask, col, 0).reshape(bt_total)
    sidx = jnp.stack([row, col])

    coef = jnp.where(mask.reshape(bt_total), g, jnp.float32(0.0))
    coefs = jnp.concatenate([coef, (1.0 - g)[None]]).astype(jnp.float32)

    e_blk = 256
    eb_total = embed_dim // e_blk
    x3 = x.reshape(bt_total, n_tokens, embed_dim)

    grid = (eb_total, bt_total)

    out = pl.pallas_call(
        _body,
        grid_spec=pltpu.PrefetchScalarGridSpec(
            num_scalar_prefetch=1,
            grid=grid,
            in_specs=[
                pl.BlockSpec(memory_space=pltpu.SMEM),
                pl.BlockSpec((1, n_tokens, e_blk),
                             lambda eb, bt, s: (bt, 0, eb)),
                pl.BlockSpec((n_tokens, e_blk),
                             lambda eb, bt, s: (0, eb)),
                pl.BlockSpec((1, 1, n_tokens, e_blk),
                             lambda eb, bt, s: (s[0, bt], s[1, bt], 0, eb)),
            ],
            out_specs=pl.BlockSpec((1, n_tokens, e_blk),
                                   lambda eb, bt, s: (bt, 0, eb)),
        ),
        out_shape=jax.ShapeDtypeStruct(x3.shape, x3.dtype),
    )(sidx, coefs, x3, local_pe, global_pe)

    return out.reshape(x.shape)


# speedup vs baseline: 1.0467x; 1.0467x over previous
import jax
import jax.numpy as jnp
from jax.experimental import pallas as pl
from jax.experimental.pallas import tpu as pltpu


def _body(sidx, coefs, x_ref, lpe_ref, gpe_ref, o_ref):
    bt = pl.program_id(1)
    n_bt = pl.num_programs(1)
    c = coefs[bt]
    lg = coefs[n_bt]
    o_ref[0] = x_ref[0] + lpe_ref[...] * lg + gpe_ref[0, 0] * c


def kernel(x, aspect_ratio, local_pe, global_pe, gate):
    bsz, n_tiles, n_tokens, embed_dim = x.shape
    bt_total = bsz * n_tiles

    g = jnp.tanh(gate)[0]
    t = jnp.arange(n_tiles, dtype=jnp.int32)
    h = aspect_ratio[:, 0:1]
    w = aspect_ratio[:, 1:2]
    w_safe = jnp.maximum(w, 1)
    row = (t[None, :] // w_safe).astype(jnp.int32)
    col = (t[None, :] % w_safe).astype(jnp.int32)
    mask = t[None, :] < (h * w)
    row = jnp.where(mask, row, 0).reshape(bt_total)
    col = jnp.where(mask, col, 0).reshape(bt_total)
    sidx = jnp.stack([row, col])

    coef = jnp.where(mask.reshape(bt_total), g, jnp.float32(0.0))
    coefs = jnp.concatenate([coef, (1.0 - g)[None]]).astype(jnp.float32)

    e_blk = embed_dim
    eb_total = embed_dim // e_blk
    x3 = x.reshape(bt_total, n_tokens, embed_dim)

    grid = (eb_total, bt_total)

    out = pl.pallas_call(
        _body,
        grid_spec=pltpu.PrefetchScalarGridSpec(
            num_scalar_prefetch=1,
            grid=grid,
            in_specs=[
                pl.BlockSpec(memory_space=pltpu.SMEM),
                pl.BlockSpec((1, n_tokens, e_blk),
                             lambda eb, bt, s: (bt, 0, eb)),
                pl.BlockSpec((n_tokens, e_blk),
                             lambda eb, bt, s: (0, eb)),
                pl.BlockSpec((1, 1, n_tokens, e_blk),
                             lambda eb, bt, s: (s[0, bt], s[1, bt], 0, eb)),
            ],
            out_specs=pl.BlockSpec((1, n_tokens, e_blk),
                                   lambda eb, bt, s: (bt, 0, eb)),
        ),
        out_shape=jax.ShapeDtypeStruct(x3.shape, x3.dtype),
    )(sidx, coefs, x3, local_pe, global_pe)

    return out.reshape(x.shape)


# speedup vs baseline: 3.0864x; 2.9487x over previous
import jax
import jax.numpy as jnp
from jax.experimental import pallas as pl
from jax.experimental.pallas import tpu as pltpu


def _body(sidx, coefs, x_ref, lpe_ref, gpe_ref, o_ref):
    bt = pl.program_id(0) * pl.num_programs(1) + pl.program_id(1)
    n_bt = pl.num_programs(0) * pl.num_programs(1)
    c = coefs[bt]
    lg = coefs[n_bt]
    o_ref[0, 0] = x_ref[0, 0] + lpe_ref[...] * lg + gpe_ref[0, 0] * c


def kernel(x, aspect_ratio, local_pe, global_pe, gate):
    bsz, n_tiles, n_tokens, embed_dim = x.shape
    bt_total = bsz * n_tiles

    g = jnp.tanh(gate)[0]
    t = jnp.arange(n_tiles, dtype=jnp.int32)
    h = aspect_ratio[:, 0:1]
    w = aspect_ratio[:, 1:2]
    w_safe = jnp.maximum(w, 1)
    row = (t[None, :] // w_safe).astype(jnp.int32)
    col = (t[None, :] % w_safe).astype(jnp.int32)
    mask = t[None, :] < (h * w)
    row = jnp.where(mask, row, 0).reshape(bt_total)
    col = jnp.where(mask, col, 0).reshape(bt_total)
    sidx = jnp.stack([row, col])

    coef = jnp.where(mask.reshape(bt_total), g, jnp.float32(0.0))
    coefs = jnp.concatenate([coef, (1.0 - g)[None]]).astype(jnp.float32)

    grid = (bsz, n_tiles)

    out = pl.pallas_call(
        _body,
        grid_spec=pltpu.PrefetchScalarGridSpec(
            num_scalar_prefetch=1,
            grid=grid,
            in_specs=[
                pl.BlockSpec(memory_space=pltpu.SMEM),
                pl.BlockSpec((1, 1, n_tokens, embed_dim),
                             lambda b, t, s: (b, t, 0, 0)),
                pl.BlockSpec((n_tokens, embed_dim),
                             lambda b, t, s: (0, 0)),
                pl.BlockSpec((1, 1, n_tokens, embed_dim),
                             lambda b, t, s: (s[0, b * n_tiles + t], s[1, b * n_tiles + t], 0, 0)),
            ],
            out_specs=pl.BlockSpec((1, 1, n_tokens, embed_dim),
                                   lambda b, t, s: (b, t, 0, 0)),
        ),
        out_shape=jax.ShapeDtypeStruct(x.shape, x.dtype),
    )(sidx, coefs, x, local_pe, global_pe)

    return out
